# baseline (speedup 1.0000x reference)
import functools

import jax
import jax.numpy as jnp
from jax import lax
from jax.experimental import pallas as pl
from jax.experimental.pallas import tpu as pltpu
from jax.experimental.pallas import tpu_sc as plsc

NW = 32
CK = 2000
G = 32
LSZ = CK + 2 * G


def _sc_agg_build(E, N, C):
  R = -(-N // NW)
  NCHUNK = E // CK
  assert (R * C) % 16 == 0 and E % CK == 0
  mesh = plsc.VectorSubcoreMesh(core_axis_name="c", subcore_axis_name="s")

  @functools.partial(
      pl.kernel,
      mesh=mesh,
      compiler_params=pltpu.CompilerParams(needs_layout_passes=False,
                                           use_tc_tiling_on_sc=False),
      out_type=jax.ShapeDtypeStruct((NW * R * C,), jnp.float32),
      scratch_types=[
          pltpu.VMEM((2, CK), jnp.int32),
          pltpu.VMEM((2, CK), jnp.int32),
          pltpu.VMEM((2, CK), jnp.float32),
          pltpu.VMEM((LSZ,), jnp.int32),
          pltpu.VMEM((LSZ,), jnp.int32),
          pltpu.VMEM((LSZ,), jnp.float32),
          pltpu.VMEM((G, C), jnp.float32),
          pltpu.VMEM((G, C), jnp.float32),
          pltpu.VMEM((R * C,), jnp.float32),
          pltpu.SemaphoreType.DMA,
          pltpu.SemaphoreType.DMA,
          pltpu.SemaphoreType.DMA,
          pltpu.SemaphoreType.DMA,
      ],
  )
  def agg_kernel(src_hbm, dst_hbm, w_hbm, sup_hbm, out_hbm,
                 dstb, srcb, wb, ld, ls, lw, rows0, rows1, acc,
                 isem0, isem1, gsem0, gsem1):
    wid = lax.axis_index("s") * 2 + lax.axis_index("c")
    lo = wid * R
    z16i = jnp.zeros((16,), jnp.int32)
    z16f = jnp.zeros((16,), jnp.float32)
    iota16 = lax.iota(jnp.int32, 16)

    def zero(i, carry):
      for u in range(4):
        acc[pl.ds((i * 4 + u) * 16, 16)] = z16f
      return carry
    lax.fori_loop(0, (R * C) // 64, zero, 0)

    def fire_idx(c, p, sem):
      pltpu.async_copy(dst_hbm.at[pl.ds(c * CK, CK)], dstb.at[p], sem)
      pltpu.async_copy(src_hbm.at[pl.ds(c * CK, CK)], srcb.at[p], sem)
      pltpu.async_copy(w_hbm.at[pl.ds(c * CK, CK)], wb.at[p], sem)

    def wait_idx(c, p, sem):
      pltpu.make_async_copy(dst_hbm.at[pl.ds(c * CK, CK)], dstb.at[p],
                            sem).wait()
      pltpu.make_async_copy(src_hbm.at[pl.ds(c * CK, CK)], srcb.at[p],
                            sem).wait()
      pltpu.make_async_copy(w_hbm.at[pl.ds(c * CK, CK)], wb.at[p],
                            sem).wait()

    def fire_gather(b, buf, sem):
      pltpu.async_copy(sup_hbm.at[ls.at[pl.ds(b * G, G)]], buf, sem)

    def wait_gather(b, buf, sem):
      pltpu.make_async_copy(sup_hbm.at[ls.at[pl.ds(b * G, G)]], buf,
                            sem).wait()

    def process(b, buf):
      def edge(e, carry):
        espl = z16i + (b * G + e)
        wspl = plsc.load_gather(lw, [espl])
        dspl = plsc.load_gather(ld, [espl])
        dbase = dspl * C
        el = z16i + e
        for g in range(C // 16):
          col = iota16 + g * 16
          v = plsc.load_gather(buf, [el, col])
          plsc.addupdate_scatter(acc, [dbase + col], v * wspl)
        return carry
      lax.fori_loop(0, G, edge, 0)

    fire_idx(0, 0, isem0)

    def chunk(c, carry):
      def cphase(p, isc, iso):
        wait_idx(c, p, isc)

        @pl.when(c < NCHUNK - 1)
        def _():
          fire_idx(c + 1, 1 - p, iso)

        def compact(i, pos):
          d = dstb[p, pl.ds(i * 16, 16)]
          loc = d - lo
          m = (loc >= 0) & (loc < R)
          mi = jnp.where(m, z16i + 1, z16i)
          posv = plsc.cumsum(mi) - 1 + pos
          plsc.store_scatter(ld, [posv], loc, mask=m)
          plsc.store_scatter(ls, [posv], srcb[p, pl.ds(i * 16, 16)], mask=m)
          plsc.store_scatter(lw, [posv], wb[p, pl.ds(i * 16, 16)], mask=m)
          return pos + jnp.sum(mi)
        pos = lax.fori_loop(0, CK // 16, compact, jnp.int32(0))

        for k in range(G // 16):
          ld[pl.ds(pos + k * 16, 16)] = z16i
          ls[pl.ds(pos + k * 16, 16)] = z16i
          lw[pl.ds(pos + k * 16, 16)] = z16f

        nb = (pos + (G - 1)) // G

        @pl.when(nb > 0)
        def _():
          fire_gather(0, rows0, gsem0)

        def block(b, carry2):
          def bphase(cur, oth, gsc, gso):
            wait_gather(b, cur, gsc)

            @pl.when(b < nb - 1)
            def _():
              fire_gather(b + 1, oth, gso)
            process(b, cur)

          @pl.when(b % 2 == 0)
          def _():
            bphase(rows0, rows1, gsem0, gsem1)

          @pl.when(b % 2 == 1)
          def _():
            bphase(rows1, rows0, gsem1, gsem0)
          return carry2
        lax.fori_loop(0, nb, block, 0)

      @pl.when(c % 2 == 0)
      def _():
        cphase(0, isem0, isem1)

      @pl.when(c % 2 == 1)
      def _():
        cphase(1, isem1, isem0)
      return carry
    lax.fori_loop(0, NCHUNK, chunk, 0)

    pltpu.sync_copy(acc, out_hbm.at[pl.ds(wid * (R * C), R * C)])

  return agg_kernel


def _sc_agg(src, dst, w, sup):
  E = src.shape[0]
  N, C = sup.shape
  fn = _sc_agg_build(E, N, C)
  out = fn(src, dst, w, sup)
  R = -(-N // NW)
  return out.reshape(NW * R, C)[:N]


def _tc1(x, W1, selr):
  N, F = x.shape
  H = W1.shape[1]
  BN = N // 10

  def body(x_ref, w_ref, s_ref, sup_ref, fs_ref):
    fs = jax.nn.sigmoid(s_ref[...])
    fs_ref[...] = fs
    sup_ref[...] = jnp.dot(x_ref[...] * fs, w_ref[...],
                           preferred_element_type=jnp.float32)

  return pl.pallas_call(
      body,
      grid=(N // BN,),
      in_specs=[
          pl.BlockSpec((BN, F), lambda i: (i, 0)),
          pl.BlockSpec((F, H), lambda i: (0, 0)),
          pl.BlockSpec((1, F), lambda i: (0, 0)),
      ],
      out_specs=[
          pl.BlockSpec((BN, H), lambda i: (i, 0)),
          pl.BlockSpec((1, F), lambda i: (0, 0)),
      ],
      out_shape=[
          jax.ShapeDtypeStruct((N, H), jnp.float32),
          jax.ShapeDtypeStruct((1, F), jnp.float32),
      ],
  )(x, W1, selr)


def _tc2(agg, b1, W2):
  N, H = agg.shape
  K = W2.shape[1]
  BN = N // 10

  def body(a_ref, b_ref, w_ref, e1_ref, s2_ref):
    e1 = a_ref[...] + b_ref[...]
    e1_ref[...] = e1
    s2_ref[...] = jnp.dot(jnp.maximum(e1, 0.0), w_ref[...],
                          preferred_element_type=jnp.float32)

  return pl.pallas_call(
      body,
      grid=(N // BN,),
      in_specs=[
          pl.BlockSpec((BN, H), lambda i: (i, 0)),
          pl.BlockSpec((1, H), lambda i: (0, 0)),
          pl.BlockSpec((H, K), lambda i: (0, 0)),
      ],
      out_specs=[
          pl.BlockSpec((BN, H), lambda i: (i, 0)),
          pl.BlockSpec((BN, K), lambda i: (i, 0)),
      ],
      out_shape=[
          jax.ShapeDtypeStruct((N, H), jnp.float32),
          jax.ShapeDtypeStruct((N, K), jnp.float32),
      ],
  )(agg, b1, W2)


def _tc3(agg2, b2):
  N, K = agg2.shape
  BN = N // 10

  def body(a_ref, b_ref, e2_ref, lp_ref):
    e2 = a_ref[...] + b_ref[...]
    e2_ref[...] = e2
    m = jnp.max(e2, axis=1, keepdims=True)
    lse = jnp.log(jnp.sum(jnp.exp(e2 - m), axis=1, keepdims=True)) + m
    lp_ref[...] = e2 - lse

  return pl.pallas_call(
      body,
      grid=(N // BN,),
      in_specs=[
          pl.BlockSpec((BN, K), lambda i: (i, 0)),
          pl.BlockSpec((1, K), lambda i: (0, 0)),
      ],
      out_specs=[
          pl.BlockSpec((BN, K), lambda i: (i, 0)),
          pl.BlockSpec((BN, K), lambda i: (i, 0)),
      ],
      out_shape=[
          jax.ShapeDtypeStruct((N, K), jnp.float32),
          jax.ShapeDtypeStruct((N, K), jnp.float32),
      ],
  )(agg2, b2)


def kernel(x, edge_index, adj_weight, W1, b1, sel_logits, W2, b2, temp):
  N, F = x.shape
  src = edge_index[0]
  dst = edge_index[1]
  selr = (sel_logits / temp).reshape(1, F).astype(jnp.float32)

  support, fs2 = _tc1(x, W1, selr)
  agg = _sc_agg(src, dst, adj_weight, support)
  embed1, support2 = _tc2(agg, b1.reshape(1, -1), W2)
  agg2 = _sc_agg(src, dst, adj_weight, support2)
  embed2, logp = _tc3(agg2, b2.reshape(1, -1))
  return logp, embed1, embed2, fs2.reshape(-1)


# speedup vs baseline: 1.0084x; 1.0084x over previous
import functools

import jax
import jax.numpy as jnp
from jax import lax
from jax.experimental import pallas as pl
from jax.experimental.pallas import tpu as pltpu
from jax.experimental.pallas import tpu_sc as plsc

NW = 32
CK = 2000
G = 32
LSZ = CK + 2 * G


def _sc_agg_build(E, N, C):
  R = -(-N // NW)
  NCHUNK = E // CK
  assert (R * C) % 16 == 0 and E % CK == 0
  mesh = plsc.VectorSubcoreMesh(core_axis_name="c", subcore_axis_name="s")

  @functools.partial(
      pl.kernel,
      mesh=mesh,
      compiler_params=pltpu.CompilerParams(needs_layout_passes=False,
                                           use_tc_tiling_on_sc=False),
      out_type=jax.ShapeDtypeStruct((NW * R * C,), jnp.float32),
      scratch_types=[
          pltpu.VMEM((2, CK), jnp.int32),
          pltpu.VMEM((2, CK), jnp.int32),
          pltpu.VMEM((2, CK), jnp.float32),
          pltpu.VMEM((LSZ,), jnp.int32),
          pltpu.VMEM((LSZ,), jnp.int32),
          pltpu.VMEM((LSZ,), jnp.float32),
          pltpu.VMEM((G, C), jnp.float32),
          pltpu.VMEM((G, C), jnp.float32),
          pltpu.VMEM((R * C,), jnp.float32),
          pltpu.SemaphoreType.DMA,
          pltpu.SemaphoreType.DMA,
          pltpu.SemaphoreType.DMA,
          pltpu.SemaphoreType.DMA,
      ],
  )
  def agg_kernel(src_hbm, dst_hbm, w_hbm, sup_hbm, out_hbm,
                 dstb, srcb, wb, ld, ls, lw, rows0, rows1, acc,
                 isem0, isem1, gsem0, gsem1):
    wid = lax.axis_index("s") * 2 + lax.axis_index("c")
    lo = wid * R
    z16i = jnp.zeros((16,), jnp.int32)
    z16f = jnp.zeros((16,), jnp.float32)
    iota16 = lax.iota(jnp.int32, 16)

    def zero(i, carry):
      for u in range(4):
        acc[pl.ds((i * 4 + u) * 16, 16)] = z16f
      return carry
    lax.fori_loop(0, (R * C) // 64, zero, 0)

    def fire_idx(c, p, sem):
      pltpu.async_copy(dst_hbm.at[pl.ds(c * CK, CK)], dstb.at[p], sem)
      pltpu.async_copy(src_hbm.at[pl.ds(c * CK, CK)], srcb.at[p], sem)
      pltpu.async_copy(w_hbm.at[pl.ds(c * CK, CK)], wb.at[p], sem)

    def wait_idx(c, p, sem):
      pltpu.make_async_copy(dst_hbm.at[pl.ds(c * CK, CK)], dstb.at[p],
                            sem).wait()
      pltpu.make_async_copy(src_hbm.at[pl.ds(c * CK, CK)], srcb.at[p],
                            sem).wait()
      pltpu.make_async_copy(w_hbm.at[pl.ds(c * CK, CK)], wb.at[p],
                            sem).wait()

    def fire_gather(b, buf, sem):
      pltpu.async_copy(sup_hbm.at[ls.at[pl.ds(b * G, G)]], buf, sem)

    def wait_gather(b, buf, sem):
      pltpu.make_async_copy(sup_hbm.at[ls.at[pl.ds(b * G, G)]], buf,
                            sem).wait()

    def process(b, buf):
      @plsc.parallel_loop(0, G, unroll=8)
      def _(e):
        espl = z16i + (b * G + e)
        wspl = plsc.load_gather(lw, [espl])
        dspl = plsc.load_gather(ld, [espl])
        dbase = dspl * C
        el = z16i + e
        for g in range(C // 16):
          col = iota16 + g * 16
          v = plsc.load_gather(buf, [el, col])
          plsc.addupdate_scatter(acc, [dbase + col], v * wspl)

    fire_idx(0, 0, isem0)

    def chunk(c, carry):
      def cphase(p, isc, iso):
        wait_idx(c, p, isc)

        @pl.when(c < NCHUNK - 1)
        def _():
          fire_idx(c + 1, 1 - p, iso)

        @plsc.parallel_loop(0, CK // 16, unroll=4, carry=jnp.int32(0))
        def pos(i, pos_c):
          d = dstb[p, pl.ds(i * 16, 16)]
          loc = d - lo
          m = (loc >= 0) & (loc < R)
          mi = jnp.where(m, z16i + 1, z16i)
          posv = plsc.cumsum(mi) - 1 + pos_c
          plsc.store_scatter(ld, [posv], loc, mask=m)
          plsc.store_scatter(ls, [posv], srcb[p, pl.ds(i * 16, 16)], mask=m)
          plsc.store_scatter(lw, [posv], wb[p, pl.ds(i * 16, 16)], mask=m)
          return pos_c + jnp.sum(mi)

        for k in range(G // 16):
          ld[pl.ds(pos + k * 16, 16)] = z16i
          ls[pl.ds(pos + k * 16, 16)] = z16i
          lw[pl.ds(pos + k * 16, 16)] = z16f

        nb = (pos + (G - 1)) // G

        @pl.when(nb > 0)
        def _():
          fire_gather(0, rows0, gsem0)

        def block(b, carry2):
          def bphase(cur, oth, gsc, gso):
            wait_gather(b, cur, gsc)

            @pl.when(b < nb - 1)
            def _():
              fire_gather(b + 1, oth, gso)
            process(b, cur)

          @pl.when(b % 2 == 0)
          def _():
            bphase(rows0, rows1, gsem0, gsem1)

          @pl.when(b % 2 == 1)
          def _():
            bphase(rows1, rows0, gsem1, gsem0)
          return carry2
        lax.fori_loop(0, nb, block, 0)

      @pl.when(c % 2 == 0)
      def _():
        cphase(0, isem0, isem1)

      @pl.when(c % 2 == 1)
      def _():
        cphase(1, isem1, isem0)
      return carry
    lax.fori_loop(0, NCHUNK, chunk, 0)

    pltpu.sync_copy(acc, out_hbm.at[pl.ds(wid * (R * C), R * C)])

  return agg_kernel


def _sc_agg(src, dst, w, sup):
  E = src.shape[0]
  N, C = sup.shape
  fn = _sc_agg_build(E, N, C)
  out = fn(src, dst, w, sup)
  R = -(-N // NW)
  return out.reshape(NW * R, C)[:N]


def _tc1(x, W1, selr):
  N, F = x.shape
  H = W1.shape[1]
  BN = N // 10

  def body(x_ref, w_ref, s_ref, sup_ref, fs_ref):
    fs = jax.nn.sigmoid(s_ref[...])
    fs_ref[...] = fs
    sup_ref[...] = jnp.dot(x_ref[...] * fs, w_ref[...],
                           preferred_element_type=jnp.float32)

  return pl.pallas_call(
      body,
      grid=(N // BN,),
      in_specs=[
          pl.BlockSpec((BN, F), lambda i: (i, 0)),
          pl.BlockSpec((F, H), lambda i: (0, 0)),
          pl.BlockSpec((1, F), lambda i: (0, 0)),
      ],
      out_specs=[
          pl.BlockSpec((BN, H), lambda i: (i, 0)),
          pl.BlockSpec((1, F), lambda i: (0, 0)),
      ],
      out_shape=[
          jax.ShapeDtypeStruct((N, H), jnp.float32),
          jax.ShapeDtypeStruct((1, F), jnp.float32),
      ],
  )(x, W1, selr)


def _tc2(agg, b1, W2):
  N, H = agg.shape
  K = W2.shape[1]
  BN = N // 10

  def body(a_ref, b_ref, w_ref, e1_ref, s2_ref):
    e1 = a_ref[...] + b_ref[...]
    e1_ref[...] = e1
    s2_ref[...] = jnp.dot(jnp.maximum(e1, 0.0), w_ref[...],
                          preferred_element_type=jnp.float32)

  return pl.pallas_call(
      body,
      grid=(N // BN,),
      in_specs=[
          pl.BlockSpec((BN, H), lambda i: (i, 0)),
          pl.BlockSpec((1, H), lambda i: (0, 0)),
          pl.BlockSpec((H, K), lambda i: (0, 0)),
      ],
      out_specs=[
          pl.BlockSpec((BN, H), lambda i: (i, 0)),
          pl.BlockSpec((BN, K), lambda i: (i, 0)),
      ],
      out_shape=[
          jax.ShapeDtypeStruct((N, H), jnp.float32),
          jax.ShapeDtypeStruct((N, K), jnp.float32),
      ],
  )(agg, b1, W2)


def _tc3(agg2, b2):
  N, K = agg2.shape
  BN = N // 10

  def body(a_ref, b_ref, e2_ref, lp_ref):
    e2 = a_ref[...] + b_ref[...]
    e2_ref[...] = e2
    m = jnp.max(e2, axis=1, keepdims=True)
    lse = jnp.log(jnp.sum(jnp.exp(e2 - m), axis=1, keepdims=True)) + m
    lp_ref[...] = e2 - lse

  return pl.pallas_call(
      body,
      grid=(N // BN,),
      in_specs=[
          pl.BlockSpec((BN, K), lambda i: (i, 0)),
          pl.BlockSpec((1, K), lambda i: (0, 0)),
      ],
      out_specs=[
          pl.BlockSpec((BN, K), lambda i: (i, 0)),
          pl.BlockSpec((BN, K), lambda i: (i, 0)),
      ],
      out_shape=[
          jax.ShapeDtypeStruct((N, K), jnp.float32),
          jax.ShapeDtypeStruct((N, K), jnp.float32),
      ],
  )(agg2, b2)


def kernel(x, edge_index, adj_weight, W1, b1, sel_logits, W2, b2, temp):
  N, F = x.shape
  src = edge_index[0]
  dst = edge_index[1]
  selr = (sel_logits / temp).reshape(1, F).astype(jnp.float32)

  support, fs2 = _tc1(x, W1, selr)
  agg = _sc_agg(src, dst, adj_weight, support)
  embed1, support2 = _tc2(agg, b1.reshape(1, -1), W2)
  agg2 = _sc_agg(src, dst, adj_weight, support2)
  embed2, logp = _tc3(agg2, b2.reshape(1, -1))
  return logp, embed1, embed2, fs2.reshape(-1)


# speedup vs baseline: 1.0102x; 1.0018x over previous
import functools

import jax
import jax.numpy as jnp
from jax import lax
from jax.experimental import pallas as pl
from jax.experimental.pallas import tpu as pltpu
from jax.experimental.pallas import tpu_sc as plsc

NW = 32
CK = 2000
G = 32
LSZ = CK + 2 * G


def _sc_agg_build(E, N, C):
  R = -(-N // NW)
  NCHUNK = E // CK
  assert (R * C) % 16 == 0 and E % CK == 0
  mesh = plsc.VectorSubcoreMesh(core_axis_name="c", subcore_axis_name="s")

  @functools.partial(
      pl.kernel,
      mesh=mesh,
      compiler_params=pltpu.CompilerParams(needs_layout_passes=False,
                                           use_tc_tiling_on_sc=False),
      out_type=jax.ShapeDtypeStruct((NW * R * C,), jnp.float32),
      scratch_types=[
          pltpu.VMEM((2, CK), jnp.int32),
          pltpu.VMEM((2, CK), jnp.int32),
          pltpu.VMEM((2, CK), jnp.float32),
          pltpu.VMEM((LSZ,), jnp.int32),
          pltpu.VMEM((LSZ,), jnp.int32),
          pltpu.VMEM((LSZ,), jnp.float32),
          pltpu.VMEM((G, C), jnp.float32),
          pltpu.VMEM((G, C), jnp.float32),
          pltpu.VMEM((R * C,), jnp.float32),
          pltpu.SemaphoreType.DMA,
          pltpu.SemaphoreType.DMA,
          pltpu.SemaphoreType.DMA,
          pltpu.SemaphoreType.DMA,
      ],
  )
  def agg_kernel(src_hbm, dst_hbm, w_hbm, sup_hbm, out_hbm,
                 dstb, srcb, wb, ld, ls, lw, rows0, rows1, acc,
                 isem0, isem1, gsem0, gsem1):
    wid = lax.axis_index("s") * 2 + lax.axis_index("c")
    lo = wid * R
    z16i = jnp.zeros((16,), jnp.int32)
    z16f = jnp.zeros((16,), jnp.float32)
    iota16 = lax.iota(jnp.int32, 16)

    def zero(i, carry):
      for u in range(4):
        acc[pl.ds((i * 4 + u) * 16, 16)] = z16f
      return carry
    lax.fori_loop(0, (R * C) // 64, zero, 0)

    def fire_idx(c, p, sem):
      pltpu.async_copy(dst_hbm.at[pl.ds(c * CK, CK)], dstb.at[p], sem)
      pltpu.async_copy(src_hbm.at[pl.ds(c * CK, CK)], srcb.at[p], sem)
      pltpu.async_copy(w_hbm.at[pl.ds(c * CK, CK)], wb.at[p], sem)

    def wait_idx(c, p, sem):
      pltpu.make_async_copy(dst_hbm.at[pl.ds(c * CK, CK)], dstb.at[p],
                            sem).wait()
      pltpu.make_async_copy(src_hbm.at[pl.ds(c * CK, CK)], srcb.at[p],
                            sem).wait()
      pltpu.make_async_copy(w_hbm.at[pl.ds(c * CK, CK)], wb.at[p],
                            sem).wait()

    def fire_gather(b, buf, sem):
      pltpu.async_copy(sup_hbm.at[ls.at[pl.ds(b * G, G)]], buf, sem)

    def wait_gather(b, buf, sem):
      pltpu.make_async_copy(sup_hbm.at[ls.at[pl.ds(b * G, G)]], buf,
                            sem).wait()

    def process(b, buf):
      @plsc.parallel_loop(0, G, unroll=8)
      def _(e):
        espl = z16i + (b * G + e)
        wspl = plsc.load_gather(lw, [espl])
        dspl = plsc.load_gather(ld, [espl])
        idx = dspl * C + iota16
        for g in range(C // 16):
          v = buf[e, pl.ds(g * 16, 16)]
          plsc.addupdate_scatter(acc, [idx], v * wspl)
          if g < C // 16 - 1:
            idx = idx + 16

    fire_idx(0, 0, isem0)

    def chunk(c, carry):
      def cphase(p, isc, iso):
        wait_idx(c, p, isc)

        @pl.when(c < NCHUNK - 1)
        def _():
          fire_idx(c + 1, 1 - p, iso)

        @plsc.parallel_loop(0, CK // 16, unroll=4, carry=jnp.int32(0))
        def pos(i, pos_c):
          d = dstb[p, pl.ds(i * 16, 16)]
          loc = d - lo
          m = (loc >= 0) & (loc < R)
          mi = jnp.where(m, z16i + 1, z16i)
          posv = plsc.cumsum(mi) - 1 + pos_c
          plsc.store_scatter(ld, [posv], loc, mask=m)
          plsc.store_scatter(ls, [posv], srcb[p, pl.ds(i * 16, 16)], mask=m)
          plsc.store_scatter(lw, [posv], wb[p, pl.ds(i * 16, 16)], mask=m)
          return pos_c + jnp.sum(mi)

        for k in range(G // 16):
          ld[pl.ds(pos + k * 16, 16)] = z16i
          ls[pl.ds(pos + k * 16, 16)] = z16i
          lw[pl.ds(pos + k * 16, 16)] = z16f

        nb = (pos + (G - 1)) // G

        @pl.when(nb > 0)
        def _():
          fire_gather(0, rows0, gsem0)

        def block(b, carry2):
          def bphase(cur, oth, gsc, gso):
            wait_gather(b, cur, gsc)

            @pl.when(b < nb - 1)
            def _():
              fire_gather(b + 1, oth, gso)
            process(b, cur)

          @pl.when(b % 2 == 0)
          def _():
            bphase(rows0, rows1, gsem0, gsem1)

          @pl.when(b % 2 == 1)
          def _():
            bphase(rows1, rows0, gsem1, gsem0)
          return carry2
        lax.fori_loop(0, nb, block, 0)

      @pl.when(c % 2 == 0)
      def _():
        cphase(0, isem0, isem1)

      @pl.when(c % 2 == 1)
      def _():
        cphase(1, isem1, isem0)
      return carry
    lax.fori_loop(0, NCHUNK, chunk, 0)

    pltpu.sync_copy(acc, out_hbm.at[pl.ds(wid * (R * C), R * C)])

  return agg_kernel


def _sc_agg(src, dst, w, sup):
  E = src.shape[0]
  N, C = sup.shape
  fn = _sc_agg_build(E, N, C)
  out = fn(src, dst, w, sup)
  R = -(-N // NW)
  return out.reshape(NW * R, C)[:N]


def _tc1(x, W1, selr):
  N, F = x.shape
  H = W1.shape[1]
  BN = N // 10

  def body(x_ref, w_ref, s_ref, sup_ref, fs_ref):
    fs = jax.nn.sigmoid(s_ref[...])
    fs_ref[...] = fs
    sup_ref[...] = jnp.dot(x_ref[...] * fs, w_ref[...],
                           preferred_element_type=jnp.float32)

  return pl.pallas_call(
      body,
      grid=(N // BN,),
      in_specs=[
          pl.BlockSpec((BN, F), lambda i: (i, 0)),
          pl.BlockSpec((F, H), lambda i: (0, 0)),
          pl.BlockSpec((1, F), lambda i: (0, 0)),
      ],
      out_specs=[
          pl.BlockSpec((BN, H), lambda i: (i, 0)),
          pl.BlockSpec((1, F), lambda i: (0, 0)),
      ],
      out_shape=[
          jax.ShapeDtypeStruct((N, H), jnp.float32),
          jax.ShapeDtypeStruct((1, F), jnp.float32),
      ],
  )(x, W1, selr)


def _tc2(agg, b1, W2):
  N, H = agg.shape
  K = W2.shape[1]
  BN = N // 10

  def body(a_ref, b_ref, w_ref, e1_ref, s2_ref):
    e1 = a_ref[...] + b_ref[...]
    e1_ref[...] = e1
    s2_ref[...] = jnp.dot(jnp.maximum(e1, 0.0), w_ref[...],
                          preferred_element_type=jnp.float32)

  return pl.pallas_call(
      body,
      grid=(N // BN,),
      in_specs=[
          pl.BlockSpec((BN, H), lambda i: (i, 0)),
          pl.BlockSpec((1, H), lambda i: (0, 0)),
          pl.BlockSpec((H, K), lambda i: (0, 0)),
      ],
      out_specs=[
          pl.BlockSpec((BN, H), lambda i: (i, 0)),
          pl.BlockSpec((BN, K), lambda i: (i, 0)),
      ],
      out_shape=[
          jax.ShapeDtypeStruct((N, H), jnp.float32),
          jax.ShapeDtypeStruct((N, K), jnp.float32),
      ],
  )(agg, b1, W2)


def _tc3(agg2, b2):
  N, K = agg2.shape
  BN = N // 10

  def body(a_ref, b_ref, e2_ref, lp_ref):
    e2 = a_ref[...] + b_ref[...]
    e2_ref[...] = e2
    m = jnp.max(e2, axis=1, keepdims=True)
    lse = jnp.log(jnp.sum(jnp.exp(e2 - m), axis=1, keepdims=True)) + m
    lp_ref[...] = e2 - lse

  return pl.pallas_call(
      body,
      grid=(N // BN,),
      in_specs=[
          pl.BlockSpec((BN, K), lambda i: (i, 0)),
          pl.BlockSpec((1, K), lambda i: (0, 0)),
      ],
      out_specs=[
          pl.BlockSpec((BN, K), lambda i: (i, 0)),
          pl.BlockSpec((BN, K), lambda i: (i, 0)),
      ],
      out_shape=[
          jax.ShapeDtypeStruct((N, K), jnp.float32),
          jax.ShapeDtypeStruct((N, K), jnp.float32),
      ],
  )(agg2, b2)


def kernel(x, edge_index, adj_weight, W1, b1, sel_logits, W2, b2, temp):
  N, F = x.shape
  src = edge_index[0]
  dst = edge_index[1]
  selr = (sel_logits / temp).reshape(1, F).astype(jnp.float32)

  support, fs2 = _tc1(x, W1, selr)
  agg = _sc_agg(src, dst, adj_weight, support)
  embed1, support2 = _tc2(agg, b1.reshape(1, -1), W2)
  agg2 = _sc_agg(src, dst, adj_weight, support2)
  embed2, logp = _tc3(agg2, b2.reshape(1, -1))
  return logp, embed1, embed2, fs2.reshape(-1)


# speedup vs baseline: 2.5930x; 2.5668x over previous
import functools

import jax
import jax.numpy as jnp
from jax import lax
from jax.experimental import pallas as pl
from jax.experimental.pallas import tpu as pltpu
from jax.experimental.pallas import tpu_sc as plsc

NW = 32
CK = 2000
G = 32
LSZ = CK + 2 * G


def _sc_agg_build(E, N, C):
  R = -(-N // NW)
  NCHUNK = E // CK
  assert (R * C) % 16 == 0 and E % CK == 0
  mesh = plsc.VectorSubcoreMesh(core_axis_name="c", subcore_axis_name="s")

  @functools.partial(
      pl.kernel,
      mesh=mesh,
      compiler_params=pltpu.CompilerParams(needs_layout_passes=False,
                                           use_tc_tiling_on_sc=False),
      out_type=jax.ShapeDtypeStruct((NW * R * C,), jnp.float32),
      scratch_types=[
          pltpu.VMEM((2, CK), jnp.int32),
          pltpu.VMEM((2, CK), jnp.int32),
          pltpu.VMEM((2, CK), jnp.float32),
          pltpu.VMEM((LSZ,), jnp.int32),
          pltpu.VMEM((LSZ,), jnp.int32),
          pltpu.VMEM((LSZ,), jnp.float32),
          pltpu.VMEM((G, C), jnp.float32),
          pltpu.VMEM((G, C), jnp.float32),
          pltpu.VMEM((R * C,), jnp.float32),
          pltpu.SemaphoreType.DMA,
          pltpu.SemaphoreType.DMA,
          pltpu.SemaphoreType.DMA,
          pltpu.SemaphoreType.DMA,
      ],
  )
  def agg_kernel(src_hbm, dst_hbm, w_hbm, sup_hbm, out_hbm,
                 dstb, srcb, wb, ld, ls, lw, rows0, rows1, acc,
                 isem0, isem1, gsem0, gsem1):
    wid = lax.axis_index("s") * 2 + lax.axis_index("c")
    lo = wid * R
    z16i = jnp.zeros((16,), jnp.int32)
    z16f = jnp.zeros((16,), jnp.float32)
    iota16 = lax.iota(jnp.int32, 16)

    def zero(i, carry):
      for u in range(4):
        acc[pl.ds((i * 4 + u) * 16, 16)] = z16f
      return carry
    lax.fori_loop(0, (R * C) // 64, zero, 0)

    def fire_idx(c, p, sem):
      pltpu.async_copy(dst_hbm.at[pl.ds(c * CK, CK)], dstb.at[p], sem)
      pltpu.async_copy(src_hbm.at[pl.ds(c * CK, CK)], srcb.at[p], sem)
      pltpu.async_copy(w_hbm.at[pl.ds(c * CK, CK)], wb.at[p], sem)

    def wait_idx(c, p, sem):
      pltpu.make_async_copy(dst_hbm.at[pl.ds(c * CK, CK)], dstb.at[p],
                            sem).wait()
      pltpu.make_async_copy(src_hbm.at[pl.ds(c * CK, CK)], srcb.at[p],
                            sem).wait()
      pltpu.make_async_copy(w_hbm.at[pl.ds(c * CK, CK)], wb.at[p],
                            sem).wait()

    def fire_gather(b, buf, sem):
      pltpu.async_copy(sup_hbm.at[ls.at[pl.ds(b * G, G)]], buf, sem)

    def wait_gather(b, buf, sem):
      pltpu.make_async_copy(sup_hbm.at[ls.at[pl.ds(b * G, G)]], buf,
                            sem).wait()

    def process(b, buf):
      @plsc.parallel_loop(0, G, unroll=8)
      def _(e):
        espl = z16i + (b * G + e)
        wspl = plsc.load_gather(lw, [espl])
        dspl = plsc.load_gather(ld, [espl])
        idx = dspl * C + iota16
        for g in range(C // 16):
          v = buf[e, pl.ds(g * 16, 16)]
          plsc.addupdate_scatter(acc, [idx], v * wspl)
          if g < C // 16 - 1:
            idx = idx + 16

    fire_idx(0, 0, isem0)

    def chunk(c, carry):
      def cphase(p, isc, iso):
        wait_idx(c, p, isc)

        @pl.when(c < NCHUNK - 1)
        def _():
          fire_idx(c + 1, 1 - p, iso)

        @plsc.parallel_loop(0, CK // 16, unroll=4, carry=jnp.int32(0))
        def pos(i, pos_c):
          d = dstb[p, pl.ds(i * 16, 16)]
          loc = d - lo
          m = (loc >= 0) & (loc < R)
          mi = jnp.where(m, z16i + 1, z16i)
          posv = plsc.cumsum(mi) - 1 + pos_c
          plsc.store_scatter(ld, [posv], loc, mask=m)
          plsc.store_scatter(ls, [posv], srcb[p, pl.ds(i * 16, 16)], mask=m)
          plsc.store_scatter(lw, [posv], wb[p, pl.ds(i * 16, 16)], mask=m)
          return pos_c + jnp.sum(mi)

        padsrc = wid * 32 + iota16
        for k in range(G // 16):
          ld[pl.ds(pos + k * 16, 16)] = z16i
          ls[pl.ds(pos + k * 16, 16)] = padsrc + k * 16
          lw[pl.ds(pos + k * 16, 16)] = z16f

        nb = (pos + (G - 1)) // G

        @pl.when(nb > 0)
        def _():
          fire_gather(0, rows0, gsem0)

        def block(b, carry2):
          def bphase(cur, oth, gsc, gso):
            wait_gather(b, cur, gsc)

            @pl.when(b < nb - 1)
            def _():
              fire_gather(b + 1, oth, gso)
            process(b, cur)

          @pl.when(b % 2 == 0)
          def _():
            bphase(rows0, rows1, gsem0, gsem1)

          @pl.when(b % 2 == 1)
          def _():
            bphase(rows1, rows0, gsem1, gsem0)
          return carry2
        lax.fori_loop(0, nb, block, 0)

      @pl.when(c % 2 == 0)
      def _():
        cphase(0, isem0, isem1)

      @pl.when(c % 2 == 1)
      def _():
        cphase(1, isem1, isem0)
      return carry
    lax.fori_loop(0, NCHUNK, chunk, 0)

    pltpu.sync_copy(acc, out_hbm.at[pl.ds(wid * (R * C), R * C)])

  return agg_kernel


def _sc_agg(src, dst, w, sup):
  E = src.shape[0]
  N, C = sup.shape
  fn = _sc_agg_build(E, N, C)
  out = fn(src, dst, w, sup)
  R = -(-N // NW)
  return out.reshape(NW * R, C)[:N]


def _tc1(x, W1, selr):
  N, F = x.shape
  H = W1.shape[1]
  BN = N // 10

  def body(x_ref, w_ref, s_ref, sup_ref, fs_ref):
    fs = jax.nn.sigmoid(s_ref[...])
    fs_ref[...] = fs
    sup_ref[...] = jnp.dot(x_ref[...] * fs, w_ref[...],
                           preferred_element_type=jnp.float32)

  return pl.pallas_call(
      body,
      grid=(N // BN,),
      in_specs=[
          pl.BlockSpec((BN, F), lambda i: (i, 0)),
          pl.BlockSpec((F, H), lambda i: (0, 0)),
          pl.BlockSpec((1, F), lambda i: (0, 0)),
      ],
      out_specs=[
          pl.BlockSpec((BN, H), lambda i: (i, 0)),
          pl.BlockSpec((1, F), lambda i: (0, 0)),
      ],
      out_shape=[
          jax.ShapeDtypeStruct((N, H), jnp.float32),
          jax.ShapeDtypeStruct((1, F), jnp.float32),
      ],
  )(x, W1, selr)


def _tc2(agg, b1, W2):
  N, H = agg.shape
  K = W2.shape[1]
  BN = N // 10

  def body(a_ref, b_ref, w_ref, e1_ref, s2_ref):
    e1 = a_ref[...] + b_ref[...]
    e1_ref[...] = e1
    s2_ref[...] = jnp.dot(jnp.maximum(e1, 0.0), w_ref[...],
                          preferred_element_type=jnp.float32)

  return pl.pallas_call(
      body,
      grid=(N // BN,),
      in_specs=[
          pl.BlockSpec((BN, H), lambda i: (i, 0)),
          pl.BlockSpec((1, H), lambda i: (0, 0)),
          pl.BlockSpec((H, K), lambda i: (0, 0)),
      ],
      out_specs=[
          pl.BlockSpec((BN, H), lambda i: (i, 0)),
          pl.BlockSpec((BN, K), lambda i: (i, 0)),
      ],
      out_shape=[
          jax.ShapeDtypeStruct((N, H), jnp.float32),
          jax.ShapeDtypeStruct((N, K), jnp.float32),
      ],
  )(agg, b1, W2)


def _tc3(agg2, b2):
  N, K = agg2.shape
  BN = N // 10

  def body(a_ref, b_ref, e2_ref, lp_ref):
    e2 = a_ref[...] + b_ref[...]
    e2_ref[...] = e2
    m = jnp.max(e2, axis=1, keepdims=True)
    lse = jnp.log(jnp.sum(jnp.exp(e2 - m), axis=1, keepdims=True)) + m
    lp_ref[...] = e2 - lse

  return pl.pallas_call(
      body,
      grid=(N // BN,),
      in_specs=[
          pl.BlockSpec((BN, K), lambda i: (i, 0)),
          pl.BlockSpec((1, K), lambda i: (0, 0)),
      ],
      out_specs=[
          pl.BlockSpec((BN, K), lambda i: (i, 0)),
          pl.BlockSpec((BN, K), lambda i: (i, 0)),
      ],
      out_shape=[
          jax.ShapeDtypeStruct((N, K), jnp.float32),
          jax.ShapeDtypeStruct((N, K), jnp.float32),
      ],
  )(agg2, b2)


def kernel(x, edge_index, adj_weight, W1, b1, sel_logits, W2, b2, temp):
  N, F = x.shape
  src = edge_index[0]
  dst = edge_index[1]
  selr = (sel_logits / temp).reshape(1, F).astype(jnp.float32)

  support, fs2 = _tc1(x, W1, selr)
  agg = _sc_agg(src, dst, adj_weight, support)
  embed1, support2 = _tc2(agg, b1.reshape(1, -1), W2)
  agg2 = _sc_agg(src, dst, adj_weight, support2)
  embed2, logp = _tc3(agg2, b2.reshape(1, -1))
  return logp, embed1, embed2, fs2.reshape(-1)


# speedup vs baseline: 2.9753x; 1.1474x over previous
import functools

import jax
import jax.numpy as jnp
from jax import lax
from jax.experimental import pallas as pl
from jax.experimental.pallas import tpu as pltpu
from jax.experimental.pallas import tpu_sc as plsc

NW = 32
CK = 2000
G = 32
LSZ = CK + 2 * G


def _sc_agg_build(E, N, C):
  R = -(-N // NW)
  NCHUNK = E // CK
  assert (R * C) % 16 == 0 and E % CK == 0
  mesh = plsc.VectorSubcoreMesh(core_axis_name="c", subcore_axis_name="s")

  @functools.partial(
      pl.kernel,
      mesh=mesh,
      compiler_params=pltpu.CompilerParams(needs_layout_passes=False,
                                           use_tc_tiling_on_sc=False),
      out_type=jax.ShapeDtypeStruct((NW * R * C,), jnp.float32),
      scratch_types=[
          pltpu.VMEM((2, CK), jnp.int32),
          pltpu.VMEM((2, CK), jnp.int32),
          pltpu.VMEM((2, CK), jnp.float32),
          pltpu.VMEM((LSZ,), jnp.int32),
          pltpu.VMEM((LSZ,), jnp.int32),
          pltpu.VMEM((LSZ,), jnp.float32),
          pltpu.VMEM((G, C), jnp.bfloat16),
          pltpu.VMEM((G, C), jnp.bfloat16),
          pltpu.VMEM((R * C,), jnp.float32),
          pltpu.SemaphoreType.DMA,
          pltpu.SemaphoreType.DMA,
          pltpu.SemaphoreType.DMA,
          pltpu.SemaphoreType.DMA,
      ],
  )
  def agg_kernel(src_hbm, dst_hbm, w_hbm, sup_hbm, out_hbm,
                 dstb, srcb, wb, ld, ls, lw, rows0, rows1, acc,
                 isem0, isem1, gsem0, gsem1):
    wid = lax.axis_index("s") * 2 + lax.axis_index("c")
    lo = wid * R
    z16i = jnp.zeros((16,), jnp.int32)
    z16f = jnp.zeros((16,), jnp.float32)
    iota16 = lax.iota(jnp.int32, 16)

    def zero(i, carry):
      for u in range(4):
        acc[pl.ds((i * 4 + u) * 16, 16)] = z16f
      return carry
    lax.fori_loop(0, (R * C) // 64, zero, 0)

    def fire_idx(c, p, sem):
      pltpu.async_copy(dst_hbm.at[pl.ds(c * CK, CK)], dstb.at[p], sem)
      pltpu.async_copy(src_hbm.at[pl.ds(c * CK, CK)], srcb.at[p], sem)
      pltpu.async_copy(w_hbm.at[pl.ds(c * CK, CK)], wb.at[p], sem)

    def wait_idx(c, p, sem):
      pltpu.make_async_copy(dst_hbm.at[pl.ds(c * CK, CK)], dstb.at[p],
                            sem).wait()
      pltpu.make_async_copy(src_hbm.at[pl.ds(c * CK, CK)], srcb.at[p],
                            sem).wait()
      pltpu.make_async_copy(w_hbm.at[pl.ds(c * CK, CK)], wb.at[p],
                            sem).wait()

    def fire_gather(b, buf, sem):
      pltpu.async_copy(sup_hbm.at[ls.at[pl.ds(b * G, G)]], buf, sem)

    def wait_gather(b, buf, sem):
      pltpu.make_async_copy(sup_hbm.at[ls.at[pl.ds(b * G, G)]], buf,
                            sem).wait()

    iota2 = lax.iota(jnp.int32, 16) * 2

    def process(b, buf):
      @plsc.parallel_loop(0, G, unroll=8)
      def _(e):
        espl = z16i + (b * G + e)
        wspl = plsc.load_gather(lw, [espl])
        dspl = plsc.load_gather(ld, [espl])
        idxe = dspl * C + iota2
        for g in range(C // 32):
          v32 = buf[e, pl.ds(g * 32, 32)]
          va, vb = plsc.unpack(v32, format=plsc.PackFormat.INTERLEAVED)
          plsc.addupdate_scatter(acc, [idxe], va * wspl)
          plsc.addupdate_scatter(acc, [idxe + 1], vb * wspl)
          if g < C // 32 - 1:
            idxe = idxe + 32

    fire_idx(0, 0, isem0)

    def chunk(c, carry):
      def cphase(p, isc, iso):
        wait_idx(c, p, isc)

        @pl.when(c < NCHUNK - 1)
        def _():
          fire_idx(c + 1, 1 - p, iso)

        @plsc.parallel_loop(0, CK // 16, unroll=4, carry=jnp.int32(0))
        def pos(i, pos_c):
          d = dstb[p, pl.ds(i * 16, 16)]
          loc = d - lo
          m = (loc >= 0) & (loc < R)
          mi = jnp.where(m, z16i + 1, z16i)
          posv = plsc.cumsum(mi) - 1 + pos_c
          plsc.store_scatter(ld, [posv], loc, mask=m)
          plsc.store_scatter(ls, [posv], srcb[p, pl.ds(i * 16, 16)], mask=m)
          plsc.store_scatter(lw, [posv], wb[p, pl.ds(i * 16, 16)], mask=m)
          return pos_c + jnp.sum(mi)

        padsrc = wid * 32 + iota16
        for k in range(G // 16):
          ld[pl.ds(pos + k * 16, 16)] = z16i
          ls[pl.ds(pos + k * 16, 16)] = padsrc + k * 16
          lw[pl.ds(pos + k * 16, 16)] = z16f

        nb = (pos + (G - 1)) // G

        @pl.when(nb > 0)
        def _():
          fire_gather(0, rows0, gsem0)

        def block(b, carry2):
          def bphase(cur, oth, gsc, gso):
            wait_gather(b, cur, gsc)

            @pl.when(b < nb - 1)
            def _():
              fire_gather(b + 1, oth, gso)
            process(b, cur)

          @pl.when(b % 2 == 0)
          def _():
            bphase(rows0, rows1, gsem0, gsem1)

          @pl.when(b % 2 == 1)
          def _():
            bphase(rows1, rows0, gsem1, gsem0)
          return carry2
        lax.fori_loop(0, nb, block, 0)

      @pl.when(c % 2 == 0)
      def _():
        cphase(0, isem0, isem1)

      @pl.when(c % 2 == 1)
      def _():
        cphase(1, isem1, isem0)
      return carry
    lax.fori_loop(0, NCHUNK, chunk, 0)

    pltpu.sync_copy(acc, out_hbm.at[pl.ds(wid * (R * C), R * C)])

  return agg_kernel


def _sc_agg(src, dst, w, sup):
  E = src.shape[0]
  N, C = sup.shape
  fn = _sc_agg_build(E, N, C)
  out = fn(src, dst, w, sup)
  R = -(-N // NW)
  return out.reshape(NW * R, C)[:N]


def _tc1(x, W1, selr):
  N, F = x.shape
  H = W1.shape[1]
  BN = N // 5

  def body(x_ref, w_ref, s_ref, sup_ref, fs_ref):
    fs = jax.nn.sigmoid(s_ref[...])
    fs_ref[...] = fs
    sup_ref[...] = jnp.dot(x_ref[...] * fs, w_ref[...],
                           preferred_element_type=jnp.float32
                           ).astype(jnp.bfloat16)

  return pl.pallas_call(
      body,
      grid=(N // BN,),
      in_specs=[
          pl.BlockSpec((BN, F), lambda i: (i, 0)),
          pl.BlockSpec((F, H), lambda i: (0, 0)),
          pl.BlockSpec((1, F), lambda i: (0, 0)),
      ],
      out_specs=[
          pl.BlockSpec((BN, H), lambda i: (i, 0)),
          pl.BlockSpec((1, F), lambda i: (0, 0)),
      ],
      out_shape=[
          jax.ShapeDtypeStruct((N, H), jnp.bfloat16),
          jax.ShapeDtypeStruct((1, F), jnp.float32),
      ],
  )(x, W1, selr)


def _tc2(agg, b1, W2):
  N, H = agg.shape
  K = W2.shape[1]
  BN = N // 5

  def body(a_ref, b_ref, w_ref, e1_ref, s2_ref):
    e1 = a_ref[...] + b_ref[...]
    e1_ref[...] = e1
    s2_ref[...] = jnp.dot(jnp.maximum(e1, 0.0), w_ref[...],
                          preferred_element_type=jnp.float32
                          ).astype(jnp.bfloat16)

  return pl.pallas_call(
      body,
      grid=(N // BN,),
      in_specs=[
          pl.BlockSpec((BN, H), lambda i: (i, 0)),
          pl.BlockSpec((1, H), lambda i: (0, 0)),
          pl.BlockSpec((H, K), lambda i: (0, 0)),
      ],
      out_specs=[
          pl.BlockSpec((BN, H), lambda i: (i, 0)),
          pl.BlockSpec((BN, K), lambda i: (i, 0)),
      ],
      out_shape=[
          jax.ShapeDtypeStruct((N, H), jnp.float32),
          jax.ShapeDtypeStruct((N, K), jnp.bfloat16),
      ],
  )(agg, b1, W2)


def _tc3(agg2, b2):
  N, K = agg2.shape
  BN = N // 10

  def body(a_ref, b_ref, e2_ref, lp_ref):
    e2 = a_ref[...] + b_ref[...]
    e2_ref[...] = e2
    m = jnp.max(e2, axis=1, keepdims=True)
    lse = jnp.log(jnp.sum(jnp.exp(e2 - m), axis=1, keepdims=True)) + m
    lp_ref[...] = e2 - lse

  return pl.pallas_call(
      body,
      grid=(N // BN,),
      in_specs=[
          pl.BlockSpec((BN, K), lambda i: (i, 0)),
          pl.BlockSpec((1, K), lambda i: (0, 0)),
      ],
      out_specs=[
          pl.BlockSpec((BN, K), lambda i: (i, 0)),
          pl.BlockSpec((BN, K), lambda i: (i, 0)),
      ],
      out_shape=[
          jax.ShapeDtypeStruct((N, K), jnp.float32),
          jax.ShapeDtypeStruct((N, K), jnp.float32),
      ],
  )(agg2, b2)


def kernel(x, edge_index, adj_weight, W1, b1, sel_logits, W2, b2, temp):
  N, F = x.shape
  src = edge_index[0]
  dst = edge_index[1]
  selr = (sel_logits / temp).reshape(1, F).astype(jnp.float32)

  support, fs2 = _tc1(x, W1, selr)
  agg = _sc_agg(src, dst, adj_weight, support)
  embed1, support2 = _tc2(agg, b1.reshape(1, -1), W2)
  agg2 = _sc_agg(src, dst, adj_weight, support2)
  embed2, logp = _tc3(agg2, b2.reshape(1, -1))
  return logp, embed1, embed2, fs2.reshape(-1)


# speedup vs baseline: 3.2636x; 1.0969x over previous
import functools

import jax
import jax.numpy as jnp
from jax import lax
from jax.experimental import pallas as pl
from jax.experimental.pallas import tpu as pltpu
from jax.experimental.pallas import tpu_sc as plsc

NT = 16
CK = 400
G = 32
LSZ = CK + 2 * G


def _sc_agg_build(E, N, CH):
  R = N // NT
  NCHUNK = E // CK
  assert (R * CH) % 16 == 0 and E % CK == 0 and N % NT == 0
  mesh = plsc.VectorSubcoreMesh(core_axis_name="c", subcore_axis_name="s")

  @functools.partial(
      pl.kernel,
      mesh=mesh,
      compiler_params=pltpu.CompilerParams(needs_layout_passes=False,
                                           use_tc_tiling_on_sc=False),
      out_type=jax.ShapeDtypeStruct((2, NT, R * CH), jnp.float32),
      scratch_types=[
          pltpu.VMEM((2, CK), jnp.int32),
          pltpu.VMEM((2, CK), jnp.int32),
          pltpu.VMEM((2, CK), jnp.float32),
          pltpu.VMEM((LSZ,), jnp.int32),
          pltpu.VMEM((LSZ,), jnp.int32),
          pltpu.VMEM((LSZ,), jnp.float32),
          pltpu.VMEM((G, CH), jnp.bfloat16),
          pltpu.VMEM((G, CH), jnp.bfloat16),
          pltpu.VMEM((R * CH,), jnp.float32),
          pltpu.VMEM_SHARED((N, CH), jnp.bfloat16),
          pltpu.SemaphoreType.DMA,
          pltpu.SemaphoreType.DMA,
          pltpu.SemaphoreType.DMA,
          pltpu.SemaphoreType.DMA,
      ],
  )
  def agg_kernel(src_hbm, dst_hbm, w_hbm, sup_hbm, out_hbm,
                 dstb, srcb, wb, ld, ls, lw, rows0, rows1, acc, spsup,
                 isem0, isem1, gsem0, gsem1):
    cid = lax.axis_index("c")
    sid = lax.axis_index("s")
    wid = sid * 2 + cid
    C = CH
    lo = sid * R
    pltpu.sync_copy(sup_hbm.at[cid, pl.ds(sid * R, R)],
                    spsup.at[pl.ds(sid * R, R)])
    z16i = jnp.zeros((16,), jnp.int32)
    z16f = jnp.zeros((16,), jnp.float32)
    iota16 = lax.iota(jnp.int32, 16)

    def zero(i, carry):
      acc[pl.ds(i * 16, 16)] = z16f
      return carry
    lax.fori_loop(0, (R * C) // 16, zero, 0)

    def fire_idx(c, p, sem):
      pltpu.async_copy(dst_hbm.at[pl.ds(c * CK, CK)], dstb.at[p], sem)
      pltpu.async_copy(src_hbm.at[pl.ds(c * CK, CK)], srcb.at[p], sem)
      pltpu.async_copy(w_hbm.at[pl.ds(c * CK, CK)], wb.at[p], sem)

    def wait_idx(c, p, sem):
      pltpu.make_async_copy(dst_hbm.at[pl.ds(c * CK, CK)], dstb.at[p],
                            sem).wait()
      pltpu.make_async_copy(src_hbm.at[pl.ds(c * CK, CK)], srcb.at[p],
                            sem).wait()
      pltpu.make_async_copy(w_hbm.at[pl.ds(c * CK, CK)], wb.at[p],
                            sem).wait()

    def fire_gather(b, buf, sem):
      pltpu.async_copy(spsup.at[ls.at[pl.ds(b * G, G)]], buf, sem)

    def wait_gather(b, buf, sem):
      pltpu.make_async_copy(spsup.at[ls.at[pl.ds(b * G, G)]], buf,
                            sem).wait()

    iota2 = lax.iota(jnp.int32, 16) * 2

    def process(b, buf):
      @plsc.parallel_loop(0, G, unroll=8)
      def _(e):
        espl = z16i + (b * G + e)
        wspl = plsc.load_gather(lw, [espl])
        dspl = plsc.load_gather(ld, [espl])
        idxe = dspl * C + iota2
        for g in range(C // 32):
          v32 = buf[e, pl.ds(g * 32, 32)]
          va, vb = plsc.unpack(v32, format=plsc.PackFormat.INTERLEAVED)
          plsc.addupdate_scatter(acc, [idxe], va * wspl)
          plsc.addupdate_scatter(acc, [idxe + 1], vb * wspl)
          if g < C // 32 - 1:
            idxe = idxe + 32

    plsc.subcore_barrier()
    fire_idx(0, 0, isem0)

    def chunk(c, carry):
      def cphase(p, isc, iso):
        wait_idx(c, p, isc)

        @pl.when(c < NCHUNK - 1)
        def _():
          fire_idx(c + 1, 1 - p, iso)

        @plsc.parallel_loop(0, CK // 16, unroll=4, carry=jnp.int32(0))
        def pos(i, pos_c):
          d = dstb[p, pl.ds(i * 16, 16)]
          loc = d - lo
          m = (loc >= 0) & (loc < R)
          mi = jnp.where(m, z16i + 1, z16i)
          posv = plsc.cumsum(mi) - 1 + pos_c
          plsc.store_scatter(ld, [posv], loc, mask=m)
          plsc.store_scatter(ls, [posv], srcb[p, pl.ds(i * 16, 16)], mask=m)
          plsc.store_scatter(lw, [posv], wb[p, pl.ds(i * 16, 16)], mask=m)
          return pos_c + jnp.sum(mi)

        padsrc = wid * 32 + iota16
        for k in range(G // 16):
          ld[pl.ds(pos + k * 16, 16)] = z16i
          ls[pl.ds(pos + k * 16, 16)] = padsrc + k * 16
          lw[pl.ds(pos + k * 16, 16)] = z16f

        nb = (pos + (G - 1)) // G

        @pl.when(nb > 0)
        def _():
          fire_gather(0, rows0, gsem0)

        def block(b, carry2):
          def bphase(cur, oth, gsc, gso):
            wait_gather(b, cur, gsc)

            @pl.when(b < nb - 1)
            def _():
              fire_gather(b + 1, oth, gso)
            process(b, cur)

          @pl.when(b % 2 == 0)
          def _():
            bphase(rows0, rows1, gsem0, gsem1)

          @pl.when(b % 2 == 1)
          def _():
            bphase(rows1, rows0, gsem1, gsem0)
          return carry2
        lax.fori_loop(0, nb, block, 0)

      @pl.when(c % 2 == 0)
      def _():
        cphase(0, isem0, isem1)

      @pl.when(c % 2 == 1)
      def _():
        cphase(1, isem1, isem0)
      return carry
    lax.fori_loop(0, NCHUNK, chunk, 0)

    pltpu.sync_copy(acc, out_hbm.at[cid, sid])

  return agg_kernel


def _sc_agg(src, dst, w, sup):
  E = src.shape[0]
  _, N, CH = sup.shape
  fn = _sc_agg_build(E, N, CH)
  out = fn(src, dst, w, sup)
  halves = out.reshape(2, N, CH)
  return jnp.concatenate([halves[0], halves[1]], axis=1)


def _tc1(x, W1, selr):
  N, F = x.shape
  H = W1.shape[1]
  BN = N // 5

  HH = H // 2

  def body(x_ref, w_ref, s_ref, sup_ref, fs_ref):
    fs = jax.nn.sigmoid(s_ref[...])
    fs_ref[...] = fs
    res = jnp.dot(x_ref[...] * fs, w_ref[...],
                  preferred_element_type=jnp.float32).astype(jnp.bfloat16)
    sup_ref[0] = res[:, :HH]
    sup_ref[1] = res[:, HH:]

  return pl.pallas_call(
      body,
      grid=(N // BN,),
      in_specs=[
          pl.BlockSpec((BN, F), lambda i: (i, 0)),
          pl.BlockSpec((F, H), lambda i: (0, 0)),
          pl.BlockSpec((1, F), lambda i: (0, 0)),
      ],
      out_specs=[
          pl.BlockSpec((2, BN, HH), lambda i: (0, i, 0)),
          pl.BlockSpec((1, F), lambda i: (0, 0)),
      ],
      out_shape=[
          jax.ShapeDtypeStruct((2, N, HH), jnp.bfloat16),
          jax.ShapeDtypeStruct((1, F), jnp.float32),
      ],
  )(x, W1, selr)


def _tc2(agg, b1, W2):
  N, H = agg.shape
  K = W2.shape[1]
  BN = N // 5

  KH = K // 2

  def body(a_ref, b_ref, w_ref, e1_ref, s2_ref):
    e1 = a_ref[...] + b_ref[...]
    e1_ref[...] = e1
    res = jnp.dot(jnp.maximum(e1, 0.0), w_ref[...],
                  preferred_element_type=jnp.float32).astype(jnp.bfloat16)
    s2_ref[0] = res[:, :KH]
    s2_ref[1] = res[:, KH:]

  return pl.pallas_call(
      body,
      grid=(N // BN,),
      in_specs=[
          pl.BlockSpec((BN, H), lambda i: (i, 0)),
          pl.BlockSpec((1, H), lambda i: (0, 0)),
          pl.BlockSpec((H, K), lambda i: (0, 0)),
      ],
      out_specs=[
          pl.BlockSpec((BN, H), lambda i: (i, 0)),
          pl.BlockSpec((2, BN, KH), lambda i: (0, i, 0)),
      ],
      out_shape=[
          jax.ShapeDtypeStruct((N, H), jnp.float32),
          jax.ShapeDtypeStruct((2, N, KH), jnp.bfloat16),
      ],
  )(agg, b1, W2)


def _tc3(agg2, b2):
  N, K = agg2.shape
  BN = N // 10

  def body(a_ref, b_ref, e2_ref, lp_ref):
    e2 = a_ref[...] + b_ref[...]
    e2_ref[...] = e2
    m = jnp.max(e2, axis=1, keepdims=True)
    lse = jnp.log(jnp.sum(jnp.exp(e2 - m), axis=1, keepdims=True)) + m
    lp_ref[...] = e2 - lse

  return pl.pallas_call(
      body,
      grid=(N // BN,),
      in_specs=[
          pl.BlockSpec((BN, K), lambda i: (i, 0)),
          pl.BlockSpec((1, K), lambda i: (0, 0)),
      ],
      out_specs=[
          pl.BlockSpec((BN, K), lambda i: (i, 0)),
          pl.BlockSpec((BN, K), lambda i: (i, 0)),
      ],
      out_shape=[
          jax.ShapeDtypeStruct((N, K), jnp.float32),
          jax.ShapeDtypeStruct((N, K), jnp.float32),
      ],
  )(agg2, b2)


def kernel(x, edge_index, adj_weight, W1, b1, sel_logits, W2, b2, temp):
  N, F = x.shape
  src = edge_index[0]
  dst = edge_index[1]
  selr = (sel_logits / temp).reshape(1, F).astype(jnp.float32)

  support, fs2 = _tc1(x, W1, selr)
  agg = _sc_agg(src, dst, adj_weight, support)
  embed1, support2 = _tc2(agg, b1.reshape(1, -1), W2)
  agg2 = _sc_agg(src, dst, adj_weight, support2)
  embed2, logp = _tc3(agg2, b2.reshape(1, -1))
  return logp, embed1, embed2, fs2.reshape(-1)


# speedup vs baseline: 3.4321x; 1.0516x over previous
import functools

import jax
import jax.numpy as jnp
from jax import lax
from jax.experimental import pallas as pl
from jax.experimental.pallas import tpu as pltpu
from jax.experimental.pallas import tpu_sc as plsc

NT = 16
CK = 400
G = 32
LSZ = CK + 2 * G


def _sc_agg_build(E, N, CH):
  R = N // NT
  NCHUNK = E // CK
  assert (R * CH) % 16 == 0 and E % CK == 0 and N % NT == 0
  mesh = plsc.VectorSubcoreMesh(core_axis_name="c", subcore_axis_name="s")

  @functools.partial(
      pl.kernel,
      mesh=mesh,
      compiler_params=pltpu.CompilerParams(needs_layout_passes=False,
                                           use_tc_tiling_on_sc=False),
      out_type=jax.ShapeDtypeStruct((2, NT, R * CH), jnp.float32),
      scratch_types=[
          pltpu.VMEM((2, CK), jnp.int32),
          pltpu.VMEM((2, CK), jnp.int32),
          pltpu.VMEM((2, CK), jnp.float32),
          pltpu.VMEM((LSZ,), jnp.int32),
          pltpu.VMEM((LSZ,), jnp.int32),
          pltpu.VMEM((LSZ,), jnp.float32),
          pltpu.VMEM((G, CH), jnp.bfloat16),
          pltpu.VMEM((G, CH), jnp.bfloat16),
          pltpu.VMEM((R * CH,), jnp.float32),
          pltpu.VMEM_SHARED((N, CH), jnp.bfloat16),
          pltpu.SemaphoreType.DMA,
          pltpu.SemaphoreType.DMA,
          pltpu.SemaphoreType.DMA,
          pltpu.SemaphoreType.DMA,
      ],
  )
  def agg_kernel(src_hbm, dst_hbm, w_hbm, sup_hbm, out_hbm,
                 dstb, srcb, wb, ld, ls, lw, rows0, rows1, acc, spsup,
                 isem0, isem1, gsem0, gsem1):
    cid = lax.axis_index("c")
    sid = lax.axis_index("s")
    wid = sid * 2 + cid
    C = CH
    lo = sid * R
    pltpu.sync_copy(sup_hbm.at[cid, pl.ds(sid * R, R)],
                    spsup.at[pl.ds(sid * R, R)])
    z16i = jnp.zeros((16,), jnp.int32)
    z16f = jnp.zeros((16,), jnp.float32)
    iota16 = lax.iota(jnp.int32, 16)

    def zero(i, carry):
      acc[pl.ds(i * 16, 16)] = z16f
      return carry
    lax.fori_loop(0, (R * C) // 16, zero, 0)

    def fire_idx(c, p, sem):
      pltpu.async_copy(dst_hbm.at[pl.ds(c * CK, CK)], dstb.at[p], sem)
      pltpu.async_copy(src_hbm.at[pl.ds(c * CK, CK)], srcb.at[p], sem)
      pltpu.async_copy(w_hbm.at[pl.ds(c * CK, CK)], wb.at[p], sem)

    def wait_idx(c, p, sem):
      pltpu.make_async_copy(dst_hbm.at[pl.ds(c * CK, CK)], dstb.at[p],
                            sem).wait()
      pltpu.make_async_copy(src_hbm.at[pl.ds(c * CK, CK)], srcb.at[p],
                            sem).wait()
      pltpu.make_async_copy(w_hbm.at[pl.ds(c * CK, CK)], wb.at[p],
                            sem).wait()

    def fire_gather(b, buf, sem):
      pltpu.async_copy(spsup.at[ls.at[pl.ds(b * G, G)]], buf, sem)

    def wait_gather(b, buf, sem):
      pltpu.make_async_copy(spsup.at[ls.at[pl.ds(b * G, G)]], buf,
                            sem).wait()

    iota2 = lax.iota(jnp.int32, 16) * 2

    def process(b, buf):
      @plsc.parallel_loop(0, G, unroll=8)
      def _(e):
        espl = z16i + (b * G + e)
        wspl = plsc.load_gather(lw, [espl])
        dspl = plsc.load_gather(ld, [espl])
        idxe = dspl * C + iota2
        for g in range(C // 32):
          v32 = buf[e, pl.ds(g * 32, 32)]
          va, vb = plsc.unpack(v32, format=plsc.PackFormat.INTERLEAVED)
          plsc.addupdate_scatter(acc, [idxe], va * wspl)
          plsc.addupdate_scatter(acc, [idxe + 1], vb * wspl)
          if g < C // 32 - 1:
            idxe = idxe + 32

    plsc.subcore_barrier()
    fire_idx(0, 0, isem0)

    def chunk(c, carry):
      def cphase(p, isc, iso):
        wait_idx(c, p, isc)

        @pl.when(c < NCHUNK - 1)
        def _():
          fire_idx(c + 1, 1 - p, iso)

        @plsc.parallel_loop(0, CK // 16, unroll=4, carry=jnp.int32(0))
        def pos(i, pos_c):
          d = dstb[p, pl.ds(i * 16, 16)]
          loc = d - lo
          m = (loc >= 0) & (loc < R)
          mi = jnp.where(m, z16i + 1, z16i)
          posv = plsc.cumsum(mi) - 1 + pos_c
          plsc.store_scatter(ld, [posv], loc, mask=m)
          plsc.store_scatter(ls, [posv], srcb[p, pl.ds(i * 16, 16)], mask=m)
          plsc.store_scatter(lw, [posv], wb[p, pl.ds(i * 16, 16)], mask=m)
          return pos_c + jnp.sum(mi)

        padsrc = wid * 32 + iota16
        for k in range(G // 16):
          ld[pl.ds(pos + k * 16, 16)] = z16i
          ls[pl.ds(pos + k * 16, 16)] = padsrc + k * 16
          lw[pl.ds(pos + k * 16, 16)] = z16f

        nb = (pos + (G - 1)) // G

        @pl.when(nb > 0)
        def _():
          fire_gather(0, rows0, gsem0)

        def block(b, carry2):
          def bphase(cur, oth, gsc, gso):
            wait_gather(b, cur, gsc)

            @pl.when(b < nb - 1)
            def _():
              fire_gather(b + 1, oth, gso)
            process(b, cur)

          @pl.when(b % 2 == 0)
          def _():
            bphase(rows0, rows1, gsem0, gsem1)

          @pl.when(b % 2 == 1)
          def _():
            bphase(rows1, rows0, gsem1, gsem0)
          return carry2
        lax.fori_loop(0, nb, block, 0)

      @pl.when(c % 2 == 0)
      def _():
        cphase(0, isem0, isem1)

      @pl.when(c % 2 == 1)
      def _():
        cphase(1, isem1, isem0)
      return carry
    lax.fori_loop(0, NCHUNK, chunk, 0)

    pltpu.sync_copy(acc, out_hbm.at[cid, sid])

  return agg_kernel


def _sc_agg(src, dst, w, sup):
  E = src.shape[0]
  _, N, CH = sup.shape
  fn = _sc_agg_build(E, N, CH)
  out = fn(src, dst, w, sup)
  return out.reshape(2, N, CH)


def _tc1(x, W1, selr):
  N, F = x.shape
  H = W1.shape[1]
  BN = N // 5

  HH = H // 2

  def body(x_ref, w_ref, s_ref, sup_ref, fs_ref):
    fs = jax.nn.sigmoid(s_ref[...])
    fs_ref[...] = fs
    res = jnp.dot(x_ref[...] * fs, w_ref[...],
                  preferred_element_type=jnp.float32).astype(jnp.bfloat16)
    sup_ref[0] = res[:, :HH]
    sup_ref[1] = res[:, HH:]

  return pl.pallas_call(
      body,
      grid=(N // BN,),
      in_specs=[
          pl.BlockSpec((BN, F), lambda i: (i, 0)),
          pl.BlockSpec((F, H), lambda i: (0, 0)),
          pl.BlockSpec((1, F), lambda i: (0, 0)),
      ],
      out_specs=[
          pl.BlockSpec((2, BN, HH), lambda i: (0, i, 0)),
          pl.BlockSpec((1, F), lambda i: (0, 0)),
      ],
      out_shape=[
          jax.ShapeDtypeStruct((2, N, HH), jnp.bfloat16),
          jax.ShapeDtypeStruct((1, F), jnp.float32),
      ],
  )(x, W1, selr)


def _tc2(agg, b1, W2):
  _, N, HH2 = agg.shape
  H = 2 * HH2
  K = W2.shape[1]
  BN = N // 5

  KH = K // 2

  def body(a_ref, b_ref, w_ref, e1_ref, s2_ref):
    e1 = jnp.concatenate([a_ref[0], a_ref[1]], axis=1) + b_ref[...]
    e1_ref[...] = e1
    res = jnp.dot(jnp.maximum(e1, 0.0), w_ref[...],
                  preferred_element_type=jnp.float32).astype(jnp.bfloat16)
    s2_ref[0] = res[:, :KH]
    s2_ref[1] = res[:, KH:]

  return pl.pallas_call(
      body,
      grid=(N // BN,),
      in_specs=[
          pl.BlockSpec((2, BN, HH2), lambda i: (0, i, 0)),
          pl.BlockSpec((1, H), lambda i: (0, 0)),
          pl.BlockSpec((H, K), lambda i: (0, 0)),
      ],
      out_specs=[
          pl.BlockSpec((BN, H), lambda i: (i, 0)),
          pl.BlockSpec((2, BN, KH), lambda i: (0, i, 0)),
      ],
      out_shape=[
          jax.ShapeDtypeStruct((N, H), jnp.float32),
          jax.ShapeDtypeStruct((2, N, KH), jnp.bfloat16),
      ],
  )(agg, b1, W2)


def _tc3(agg2, b2):
  _, N, KH2 = agg2.shape
  K = 2 * KH2
  BN = N // 10

  def body(a_ref, b_ref, e2_ref, lp_ref):
    e2 = jnp.concatenate([a_ref[0], a_ref[1]], axis=1) + b_ref[...]
    e2_ref[...] = e2
    m = jnp.max(e2, axis=1, keepdims=True)
    lse = jnp.log(jnp.sum(jnp.exp(e2 - m), axis=1, keepdims=True)) + m
    lp_ref[...] = e2 - lse

  return pl.pallas_call(
      body,
      grid=(N // BN,),
      in_specs=[
          pl.BlockSpec((2, BN, KH2), lambda i: (0, i, 0)),
          pl.BlockSpec((1, K), lambda i: (0, 0)),
      ],
      out_specs=[
          pl.BlockSpec((BN, K), lambda i: (i, 0)),
          pl.BlockSpec((BN, K), lambda i: (i, 0)),
      ],
      out_shape=[
          jax.ShapeDtypeStruct((N, K), jnp.float32),
          jax.ShapeDtypeStruct((N, K), jnp.float32),
      ],
  )(agg2, b2)


def kernel(x, edge_index, adj_weight, W1, b1, sel_logits, W2, b2, temp):
  N, F = x.shape
  src = edge_index[0]
  dst = edge_index[1]
  selr = (sel_logits / temp).reshape(1, F).astype(jnp.float32)

  support, fs2 = _tc1(x, W1, selr)
  agg = _sc_agg(src, dst, adj_weight, support)
  embed1, support2 = _tc2(agg, b1.reshape(1, -1), W2)
  agg2 = _sc_agg(src, dst, adj_weight, support2)
  embed2, logp = _tc3(agg2, b2.reshape(1, -1))
  return logp, embed1, embed2, fs2.reshape(-1)
